# Initial kernel scaffold; baseline (speedup 1.0000x reference)
import jax
import jax.numpy as jnp
from jax.experimental import pallas as pl


def kernel(x, edge_index, edge_label_index, W1, b1, W2, b2):
    raise NotImplementedError("write your pallas kernel here")


# speedup vs baseline: 5.6777x; 5.6777x over previous
import functools

import jax
import jax.numpy as jnp
from jax import lax
from jax.experimental import pallas as pl
from jax.experimental.pallas import tpu as pltpu
from jax.experimental.pallas import tpu_sc as plsc

N_NODES = 10000
N_EDGES = 320000
N_LABEL = 200000
IN_CH = 128
HID_CH = 128
OUT_CH = 64

NC = 2
NS = 16
NW = NC * NS
LANES = 16

NP = 10240
ROWS_PER_SC_SUB = NP // NS

E_RPW = 79
E_PAD = NW * E_RPW * 128
L_RPW = 50
L_PAD = NW * L_RPW * 128

_MESH = plsc.VectorSubcoreMesh(core_axis_name="c", subcore_axis_name="s")


def _zero16():
    return jnp.zeros((LANES,), jnp.float32)


def _sc_degree(dst2d):
    @functools.partial(
        pl.kernel,
        mesh=_MESH,
        out_type=jax.ShapeDtypeStruct((NC, NP, LANES), jnp.float32),
        scratch_types=[
            pltpu.VMEM((E_RPW, 128), jnp.int32),
            pltpu.VMEM((128, LANES), jnp.float32),
            pltpu.VMEM((128, LANES), jnp.float32),
            pltpu.VMEM_SHARED((NP, LANES), jnp.float32),
        ],
    )
    def k(dst_hbm, hist_hbm, idx_v, ones_v, zbuf_v, hist_sh):
        cid = lax.axis_index("c")
        sid = lax.axis_index("s")
        wid = sid * NC + cid
        one = jnp.full((LANES,), 1.0, jnp.float32)
        zero = _zero16()

        @pl.loop(0, 128)
        def _(i):
            ones_v[i, :] = one
            zbuf_v[i, :] = zero

        @pl.loop(0, ROWS_PER_SC_SUB // 128)
        def _(j):
            pltpu.sync_copy(zbuf_v,
                            hist_sh.at[pl.ds(sid * ROWS_PER_SC_SUB + j * 128, 128)])
        plsc.subcore_barrier()

        pltpu.sync_copy(dst_hbm.at[wid], idx_v)

        @pl.loop(0, E_RPW)
        def _(j):
            pltpu.sync_copy(ones_v, hist_sh.at[idx_v.at[j]], add=True)
        plsc.subcore_barrier()

        pltpu.sync_copy(hist_sh.at[pl.ds(sid * ROWS_PER_SC_SUB, ROWS_PER_SC_SUB)],
                        hist_hbm.at[cid, pl.ds(sid * ROWS_PER_SC_SUB, ROWS_PER_SC_SUB)])

    return k(dst2d)


def _sc_aggregate(table, src2d, dst2d, D):
    @functools.partial(
        pl.kernel,
        mesh=_MESH,
        out_type=jax.ShapeDtypeStruct((NC, NP, D), jnp.float32),
        scratch_types=[
            pltpu.VMEM((E_RPW, 128), jnp.int32),
            pltpu.VMEM((E_RPW, 128), jnp.int32),
            pltpu.VMEM((128, D), jnp.float32),
            pltpu.VMEM_SHARED((NP, D), jnp.float32),
            pltpu.SemaphoreType.DMA,
        ],
    )
    def k(tab_hbm, src_hbm, dst_hbm, out_hbm, si_v, di_v, rows_v, acc_sh, sem):
        cid = lax.axis_index("c")
        sid = lax.axis_index("s")
        wid = sid * NC + cid
        zero = _zero16()

        @pl.loop(0, 128)
        def _(i):
            for c in range(D // LANES):
                rows_v[i, pl.ds(c * LANES, LANES)] = zero

        @pl.loop(0, ROWS_PER_SC_SUB // 128)
        def _(j):
            pltpu.sync_copy(rows_v,
                            acc_sh.at[pl.ds(sid * ROWS_PER_SC_SUB + j * 128, 128)])
        plsc.subcore_barrier()

        pltpu.sync_copy(src_hbm.at[wid], si_v)
        pltpu.sync_copy(dst_hbm.at[wid], di_v)

        @pl.loop(0, E_RPW)
        def _(j):
            pltpu.async_copy(tab_hbm.at[si_v.at[j]], rows_v, sem).wait()
            pltpu.sync_copy(rows_v, acc_sh.at[di_v.at[j]], add=True)
        plsc.subcore_barrier()

        pltpu.sync_copy(acc_sh.at[pl.ds(sid * ROWS_PER_SC_SUB, ROWS_PER_SC_SUB)],
                        out_hbm.at[cid, pl.ds(sid * ROWS_PER_SC_SUB, ROWS_PER_SC_SUB)])

    return k(table, src2d, dst2d)


def _sc_decode_gather(z, e0_2d, e1_2d):
    @functools.partial(
        pl.kernel,
        mesh=_MESH,
        out_type=jax.ShapeDtypeStruct((2, L_PAD, HID_CH), jnp.float32),
        scratch_types=[
            pltpu.VMEM((L_RPW, 128), jnp.int32),
            pltpu.VMEM((128, HID_CH), jnp.float32),
            pltpu.SemaphoreType.DMA,
        ],
    )
    def k(z_hbm, e0_hbm, e1_hbm, out_hbm, idx_v, rows_v, sem):
        cid = lax.axis_index("c")
        sid = lax.axis_index("s")
        wid = sid * NC + cid
        for side, e_hbm in ((0, e0_hbm), (1, e1_hbm)):
            pltpu.sync_copy(e_hbm.at[wid], idx_v)

            @pl.loop(0, L_RPW)
            def _(j, side=side):
                pltpu.async_copy(z_hbm.at[idx_v.at[j]], rows_v, sem).wait()
                pltpu.sync_copy(
                    rows_v,
                    out_hbm.at[side, pl.ds(wid * L_RPW * 128 + j * 128, 128)])

    return k(z, e0_2d, e1_2d)


_BLK = 512


def _dot(a, b):
    return jnp.dot(a, b, preferred_element_type=jnp.float32,
                   precision=lax.Precision.HIGHEST)


def _tc_stage1(hist, x_pad, W1):
    def body(hist_ref, x_ref, w_ref, dis_ref, hs_ref):
        deg = hist_ref[0] + hist_ref[1] + 1.0
        dis = lax.rsqrt(deg)
        dis_ref[...] = dis
        h = _dot(x_ref[...], w_ref[...])
        hs_ref[...] = h * dis[:, 0:1]

    return pl.pallas_call(
        body,
        grid=(NP // _BLK,),
        in_specs=[
            pl.BlockSpec((NC, _BLK, LANES), lambda i: (0, i, 0)),
            pl.BlockSpec((_BLK, IN_CH), lambda i: (i, 0)),
            pl.BlockSpec((IN_CH, HID_CH), lambda i: (0, 0)),
        ],
        out_specs=[
            pl.BlockSpec((_BLK, LANES), lambda i: (i, 0)),
            pl.BlockSpec((_BLK, HID_CH), lambda i: (i, 0)),
        ],
        out_shape=[
            jax.ShapeDtypeStruct((NP, LANES), jnp.float32),
            jax.ShapeDtypeStruct((NP, HID_CH), jnp.float32),
        ],
    )(hist, x_pad, W1)


def _tc_stage2(parts1, hs1, dis, b1, W2):
    def body(p_ref, hs_ref, dis_ref, b_ref, w_ref, out_ref):
        d = dis_ref[:, 0:1]
        z1 = jnp.maximum((p_ref[0] + p_ref[1] + hs_ref[...]) * d + b_ref[...], 0.0)
        out_ref[:, :OUT_CH] = _dot(z1, w_ref[...]) * d
        out_ref[:, OUT_CH:] = jnp.zeros((_BLK, HID_CH - OUT_CH), jnp.float32)

    return pl.pallas_call(
        body,
        grid=(NP // _BLK,),
        in_specs=[
            pl.BlockSpec((NC, _BLK, HID_CH), lambda i: (0, i, 0)),
            pl.BlockSpec((_BLK, HID_CH), lambda i: (i, 0)),
            pl.BlockSpec((_BLK, LANES), lambda i: (i, 0)),
            pl.BlockSpec((1, HID_CH), lambda i: (0, 0)),
            pl.BlockSpec((HID_CH, OUT_CH), lambda i: (0, 0)),
        ],
        out_specs=pl.BlockSpec((_BLK, HID_CH), lambda i: (i, 0)),
        out_shape=jax.ShapeDtypeStruct((NP, HID_CH), jnp.float32),
    )(parts1, hs1, dis, b1, W2)


def _tc_stage3(parts2, hs2, dis, b2):
    def body(p_ref, hs_ref, dis_ref, b_ref, out_ref):
        d = dis_ref[:, 0:1]
        out_ref[...] = (p_ref[0] + p_ref[1] + hs_ref[...]) * d + b_ref[...]

    return pl.pallas_call(
        body,
        grid=(NP // _BLK,),
        in_specs=[
            pl.BlockSpec((NC, _BLK, HID_CH), lambda i: (0, i, 0)),
            pl.BlockSpec((_BLK, HID_CH), lambda i: (i, 0)),
            pl.BlockSpec((_BLK, LANES), lambda i: (i, 0)),
            pl.BlockSpec((1, HID_CH), lambda i: (0, 0)),
        ],
        out_specs=pl.BlockSpec((_BLK, HID_CH), lambda i: (i, 0)),
        out_shape=jax.ShapeDtypeStruct((NP, HID_CH), jnp.float32),
    )(parts2, hs2, dis, b2)


_DBLK = 1024


def _tc_decode_dot(a, b):
    def body(a_ref, b_ref, out_ref):
        s = jnp.sum(a_ref[...] * b_ref[...], axis=1)
        out_ref[...] = s.reshape(_DBLK // 128, 128)

    return pl.pallas_call(
        body,
        grid=(L_PAD // _DBLK,),
        in_specs=[
            pl.BlockSpec((_DBLK, HID_CH), lambda i: (i, 0)),
            pl.BlockSpec((_DBLK, HID_CH), lambda i: (i, 0)),
        ],
        out_specs=pl.BlockSpec((_DBLK // 128, 128), lambda i: (i, 0)),
        out_shape=jax.ShapeDtypeStruct((L_PAD // 128, 128), jnp.float32),
    )(a, b)


def kernel(x, edge_index, edge_label_index, W1, b1, W2, b2):
    ei = edge_index.astype(jnp.int32)
    eli = edge_label_index.astype(jnp.int32)

    epad = E_PAD - N_EDGES
    src2d = jnp.concatenate(
        [ei[0], jnp.full((epad,), N_NODES, jnp.int32)]).reshape(NW, E_RPW, 128)
    dst2d = jnp.concatenate(
        [ei[1], jnp.full((epad,), N_NODES, jnp.int32)]).reshape(NW, E_RPW, 128)

    lpad = L_PAD - N_LABEL
    e0_2d = jnp.concatenate(
        [eli[0], jnp.zeros((lpad,), jnp.int32)]).reshape(NW, L_RPW, 128)
    e1_2d = jnp.concatenate(
        [eli[1], jnp.zeros((lpad,), jnp.int32)]).reshape(NW, L_RPW, 128)

    x_pad = jnp.pad(x, ((0, NP - N_NODES), (0, 0)))

    b2p = jnp.pad(b2.reshape(1, OUT_CH), ((0, 0), (0, HID_CH - OUT_CH)))

    hist = _sc_degree(dst2d)
    dis, hs1 = _tc_stage1(hist, x_pad, W1)
    parts1 = _sc_aggregate(hs1, src2d, dst2d, HID_CH)
    hs2 = _tc_stage2(parts1, hs1, dis, b1.reshape(1, HID_CH), W2)
    parts2 = _sc_aggregate(hs2, src2d, dst2d, HID_CH)
    z = _tc_stage3(parts2, hs2, dis, b2p)
    ab = _sc_decode_gather(z, e0_2d, e1_2d)
    out2d = _tc_decode_dot(ab[0], ab[1])
    return out2d.reshape(-1)[:N_LABEL]
